# baseline (speedup 1.0000x reference)
import jax
import jax.numpy as jnp
from jax.experimental import pallas as pl
from jax.experimental.pallas import tpu as pltpu

_E = 8
_TOPK = 2
_BLK = 128
_BLKM = 256
_N = 2048
_D = 768
_F = 3072
_S = _N * _TOPK
_P = _S + _E * _BLKM
_NB = _P // _BLKM


def _gating_body(x_ref, wg_ref, bg_ref, topi_ref, w_ref):
    logits = jnp.dot(x_ref[...], wg_ref[...], preferred_element_type=jnp.float32)
    logits = logits + bg_ref[...]
    lane = jax.lax.broadcasted_iota(jnp.int32, logits.shape, 1)
    m1 = jnp.max(logits, axis=1, keepdims=True)
    i1 = jnp.min(jnp.where(logits == m1, lane, _E), axis=1, keepdims=True)
    l2 = jnp.where(lane == i1, -jnp.inf, logits)
    m2 = jnp.max(l2, axis=1, keepdims=True)
    i2 = jnp.min(jnp.where(l2 == m2, lane, _E), axis=1, keepdims=True)
    r = jnp.exp(m2 - m1)
    topi_ref[...] = jnp.concatenate([i1, i2], axis=1)
    w_ref[...] = jnp.concatenate([1.0 / (1.0 + r), r / (1.0 + r)], axis=1)


def _route(topi):
    e_flat = topi.reshape(-1).astype(jnp.int32)
    onehot = (e_flat[:, None] == jnp.arange(_E, dtype=jnp.int32)[None, :]).astype(jnp.int32)
    csum = jnp.cumsum(onehot, axis=0)
    counts = csum[-1]
    rank = jnp.take_along_axis(csum, e_flat[:, None], axis=1)[:, 0] - 1
    padded = ((counts + _BLKM - 1) // _BLKM) * _BLKM
    pad_off = jnp.concatenate([jnp.zeros((1,), jnp.int32), jnp.cumsum(padded)[:-1]])
    slot = pad_off[e_flat] + rank
    src_token = jnp.zeros((_P,), jnp.int32).at[slot].set(
        jnp.arange(_S, dtype=jnp.int32) // _TOPK)
    cumblk = jnp.cumsum(padded // _BLKM)
    bidx = jnp.arange(_NB, dtype=jnp.int32)
    block_expert = jnp.sum((bidx[:, None] >= cumblk[None, :]).astype(jnp.int32), axis=1)
    block_expert = jnp.minimum(block_expert, _E - 1)
    return slot, src_token, block_expert


def _mlp_body(be_ref, bufi_ref, start_ref, nxt_ref, st_ref,
              x_ref, w1_ref, b1_ref, w2_ref, b2_ref, ys_ref,
              xs_ref, w1buf, w2buf, sems):
    i = pl.program_id(0)
    b = bufi_ref[i]

    @pl.when(i == 0)
    def _():
        e0 = be_ref[0]
        pltpu.make_async_copy(w1_ref.at[e0], w1buf.at[0], sems.at[0, 0]).start()
        pltpu.make_async_copy(w2_ref.at[e0], w2buf.at[0], sems.at[0, 1]).start()

    @pl.when(start_ref[i] == 1)
    def _():
        nxt = nxt_ref[i]

        @pl.when(nxt >= 0)
        def _():
            nb = 1 - b
            pltpu.make_async_copy(w1_ref.at[nxt], w1buf.at[nb], sems.at[nb, 0]).start()
            pltpu.make_async_copy(w2_ref.at[nxt], w2buf.at[nb], sems.at[nb, 1]).start()

        pltpu.make_async_copy(w1_ref.at[be_ref[i]], w1buf.at[b], sems.at[b, 0]).wait()
        pltpu.make_async_copy(w2_ref.at[be_ref[i]], w2buf.at[b], sems.at[b, 1]).wait()

    g = i % 2

    def gather(base, dst):
        def body(j, carry):
            tok = st_ref[base + j]
            xs_ref[dst, pl.ds(j, 1), :] = x_ref[pl.ds(tok, 1), :]
            return carry

        jax.lax.fori_loop(0, _BLKM, body, 0, unroll=8)

    @pl.when(i == 0)
    def _():
        gather(0, 0)

    h = jnp.dot(xs_ref[g], w1buf[b], preferred_element_type=jnp.float32) + b1_ref[0]
    h = jnp.maximum(h, 0.0)
    ys_ref[...] = jnp.dot(h, w2buf[b], preferred_element_type=jnp.float32) + b2_ref[0]

    @pl.when(i + 1 < _NB)
    def _():
        gather((i + 1) * _BLKM, 1 - g)


def _combine_body(pos_ref, w_ref, ys_ref, out_ref, a_ref, b_ref):
    i = pl.program_id(0)

    def gather(j, carry):
        n = i * _BLK + j
        a_ref[pl.ds(j, 1), :] = ys_ref[pl.ds(pos_ref[2 * n], 1), :]
        b_ref[pl.ds(j, 1), :] = ys_ref[pl.ds(pos_ref[2 * n + 1], 1), :]
        return carry

    jax.lax.fori_loop(0, _BLK, gather, 0, unroll=8)
    out_ref[...] = w_ref[:, 0:1] * a_ref[...] + w_ref[:, 1:2] * b_ref[...]


def kernel(x, Wg, bg, W1, b1, W2, b2):
    topi, w = pl.pallas_call(
        _gating_body,
        grid=(4,),
        in_specs=[
            pl.BlockSpec((_N // 4, _D), lambda i: (i, 0)),
            pl.BlockSpec((_D, _E), lambda i: (0, 0)),
            pl.BlockSpec((1, _E), lambda i: (0, 0)),
        ],
        out_specs=[
            pl.BlockSpec((_N // 4, _TOPK), lambda i: (i, 0)),
            pl.BlockSpec((_N // 4, _TOPK), lambda i: (i, 0)),
        ],
        out_shape=[
            jax.ShapeDtypeStruct((_N, _TOPK), jnp.int32),
            jax.ShapeDtypeStruct((_N, _TOPK), jnp.float32),
        ],
        compiler_params=pltpu.CompilerParams(dimension_semantics=("parallel",)),
    )(x, Wg, bg.reshape(1, _E))

    slot, src_token, block_expert = _route(topi)

    is_start = jnp.concatenate(
        [jnp.ones((1,), jnp.int32),
         (block_expert[1:] != block_expert[:-1]).astype(jnp.int32)])
    run_idx = jnp.cumsum(is_start) - 1
    bufi = (run_idx % 2).astype(jnp.int32)
    eidx = jnp.arange(_E, dtype=jnp.int32)
    present = jnp.any(block_expert[None, :] == eidx[:, None], axis=1)
    cand = jnp.where(present[None, :] & (eidx[None, :] > block_expert[:, None]),
                     eidx[None, :], _E)
    nxt = jnp.min(cand, axis=1).astype(jnp.int32)
    nxt = jnp.where(nxt == _E, -1, nxt)

    ys = pl.pallas_call(
        _mlp_body,
        grid_spec=pltpu.PrefetchScalarGridSpec(
            num_scalar_prefetch=5,
            grid=(_NB,),
            in_specs=[
                pl.BlockSpec((_N, _D), lambda i, *_: (0, 0)),
                pl.BlockSpec(memory_space=pl.ANY),
                pl.BlockSpec((1, 1, _F), lambda i, be, *_: (be[i], 0, 0)),
                pl.BlockSpec(memory_space=pl.ANY),
                pl.BlockSpec((1, 1, _D), lambda i, be, *_: (be[i], 0, 0)),
            ],
            out_specs=pl.BlockSpec((_BLKM, _D), lambda i, *_: (i, 0)),
            scratch_shapes=[
                pltpu.VMEM((2, _BLKM, _D), jnp.float32),
                pltpu.VMEM((2, _D, _F), jnp.float32),
                pltpu.VMEM((2, _F, _D), jnp.float32),
                pltpu.SemaphoreType.DMA((2, 2)),
            ],
        ),
        out_shape=jax.ShapeDtypeStruct((_P, _D), jnp.float32),
        compiler_params=pltpu.CompilerParams(dimension_semantics=("arbitrary",)),
    )(block_expert, bufi, is_start, nxt, src_token,
      x, W1, b1.reshape(_E, 1, _F), W2, b2.reshape(_E, 1, _D))

    out = pl.pallas_call(
        _combine_body,
        grid_spec=pltpu.PrefetchScalarGridSpec(
            num_scalar_prefetch=1,
            grid=(_N // _BLK,),
            in_specs=[
                pl.BlockSpec((_BLK, _TOPK), lambda i, pos: (i, 0)),
                pl.BlockSpec((_P, _D), lambda i, pos: (0, 0)),
            ],
            out_specs=pl.BlockSpec((_BLK, _D), lambda i, pos: (i, 0)),
            scratch_shapes=[
                pltpu.VMEM((_BLK, _D), jnp.float32),
                pltpu.VMEM((_BLK, _D), jnp.float32),
            ],
        ),
        out_shape=jax.ShapeDtypeStruct((_N, _D), jnp.float32),
        compiler_params=pltpu.CompilerParams(dimension_semantics=("parallel",)),
    )(slot, w, ys)
    return out


# speedup vs baseline: 1.0239x; 1.0239x over previous
import jax
import jax.numpy as jnp
from jax.experimental import pallas as pl
from jax.experimental.pallas import tpu as pltpu

_E = 8
_TOPK = 2
_BLK = 128
_BLKM = 128
_N = 2048
_D = 768
_F = 3072
_S = _N * _TOPK
_P = _S + _E * _BLKM
_NB = _P // _BLKM


def _gating_body(x_ref, wg_ref, bg_ref, topi_ref, w_ref):
    logits = jnp.dot(x_ref[...], wg_ref[...], preferred_element_type=jnp.float32)
    logits = logits + bg_ref[...]
    lane = jax.lax.broadcasted_iota(jnp.int32, logits.shape, 1)
    m1 = jnp.max(logits, axis=1, keepdims=True)
    i1 = jnp.min(jnp.where(logits == m1, lane, _E), axis=1, keepdims=True)
    l2 = jnp.where(lane == i1, -jnp.inf, logits)
    m2 = jnp.max(l2, axis=1, keepdims=True)
    i2 = jnp.min(jnp.where(l2 == m2, lane, _E), axis=1, keepdims=True)
    r = jnp.exp(m2 - m1)
    topi_ref[...] = jnp.concatenate([i1, i2], axis=1)
    w_ref[...] = jnp.concatenate([1.0 / (1.0 + r), r / (1.0 + r)], axis=1)


def _route(topi):
    e_flat = topi.reshape(-1).astype(jnp.int32)
    onehot = (e_flat[:, None] == jnp.arange(_E, dtype=jnp.int32)[None, :]).astype(jnp.int32)
    csum = jnp.cumsum(onehot, axis=0)
    counts = csum[-1]
    rank = jnp.take_along_axis(csum, e_flat[:, None], axis=1)[:, 0] - 1
    padded = ((counts + _BLKM - 1) // _BLKM) * _BLKM
    pad_off = jnp.concatenate([jnp.zeros((1,), jnp.int32), jnp.cumsum(padded)[:-1]])
    slot = pad_off[e_flat] + rank
    src_token = jnp.zeros((_P,), jnp.int32).at[slot].set(
        jnp.arange(_S, dtype=jnp.int32) // _TOPK)
    cumblk = jnp.cumsum(padded // _BLKM)
    bidx = jnp.arange(_NB, dtype=jnp.int32)
    block_expert = jnp.sum((bidx[:, None] >= cumblk[None, :]).astype(jnp.int32), axis=1)
    block_expert = jnp.minimum(block_expert, _E - 1)
    return slot, src_token, block_expert


def _mlp_body(be_ref, bufi_ref, start_ref, nxt_ref, st_ref,
              x_ref, w1_ref, b1_ref, w2_ref, b2_ref, ys_ref,
              xs_ref, w1buf, w2buf, sems):
    i = pl.program_id(0)
    b = bufi_ref[i]

    @pl.when(i == 0)
    def _():
        e0 = be_ref[0]
        pltpu.make_async_copy(w1_ref.at[e0], w1buf.at[0], sems.at[0, 0]).start()
        pltpu.make_async_copy(w2_ref.at[e0], w2buf.at[0], sems.at[0, 1]).start()

    @pl.when(start_ref[i] == 1)
    def _():
        nxt = nxt_ref[i]

        @pl.when(nxt >= 0)
        def _():
            nb = 1 - b
            pltpu.make_async_copy(w1_ref.at[nxt], w1buf.at[nb], sems.at[nb, 0]).start()
            pltpu.make_async_copy(w2_ref.at[nxt], w2buf.at[nb], sems.at[nb, 1]).start()

        pltpu.make_async_copy(w1_ref.at[be_ref[i]], w1buf.at[b], sems.at[b, 0]).wait()
        pltpu.make_async_copy(w2_ref.at[be_ref[i]], w2buf.at[b], sems.at[b, 1]).wait()

    g = i % 2

    def gather(base, dst):
        def body(j, carry):
            tok = st_ref[base + j]
            xs_ref[dst, pl.ds(j, 1), :] = x_ref[pl.ds(tok, 1), :]
            return carry

        jax.lax.fori_loop(0, _BLKM, body, 0, unroll=8)

    @pl.when(i == 0)
    def _():
        gather(0, 0)

    h = jnp.dot(xs_ref[g], w1buf[b], preferred_element_type=jnp.float32) + b1_ref[0]
    h = jnp.maximum(h, 0.0)
    ys_ref[...] = jnp.dot(h, w2buf[b], preferred_element_type=jnp.float32) + b2_ref[0]

    @pl.when(i + 1 < _NB)
    def _():
        gather((i + 1) * _BLKM, 1 - g)


def _combine_body(pos_ref, w_ref, ys_ref, out_ref, a_ref, b_ref):
    i = pl.program_id(0)

    def gather(j, carry):
        n = i * _BLK + j
        a_ref[pl.ds(j, 1), :] = ys_ref[pl.ds(pos_ref[2 * n], 1), :]
        b_ref[pl.ds(j, 1), :] = ys_ref[pl.ds(pos_ref[2 * n + 1], 1), :]
        return carry

    jax.lax.fori_loop(0, _BLK, gather, 0, unroll=8)
    out_ref[...] = w_ref[:, 0:1] * a_ref[...] + w_ref[:, 1:2] * b_ref[...]


def kernel(x, Wg, bg, W1, b1, W2, b2):
    topi, w = pl.pallas_call(
        _gating_body,
        grid=(4,),
        in_specs=[
            pl.BlockSpec((_N // 4, _D), lambda i: (i, 0)),
            pl.BlockSpec((_D, _E), lambda i: (0, 0)),
            pl.BlockSpec((1, _E), lambda i: (0, 0)),
        ],
        out_specs=[
            pl.BlockSpec((_N // 4, _TOPK), lambda i: (i, 0)),
            pl.BlockSpec((_N // 4, _TOPK), lambda i: (i, 0)),
        ],
        out_shape=[
            jax.ShapeDtypeStruct((_N, _TOPK), jnp.int32),
            jax.ShapeDtypeStruct((_N, _TOPK), jnp.float32),
        ],
        compiler_params=pltpu.CompilerParams(dimension_semantics=("parallel",)),
    )(x, Wg, bg.reshape(1, _E))

    slot, src_token, block_expert = _route(topi)

    is_start = jnp.concatenate(
        [jnp.ones((1,), jnp.int32),
         (block_expert[1:] != block_expert[:-1]).astype(jnp.int32)])
    run_idx = jnp.cumsum(is_start) - 1
    bufi = (run_idx % 2).astype(jnp.int32)
    eidx = jnp.arange(_E, dtype=jnp.int32)
    present = jnp.any(block_expert[None, :] == eidx[:, None], axis=1)
    cand = jnp.where(present[None, :] & (eidx[None, :] > block_expert[:, None]),
                     eidx[None, :], _E)
    nxt = jnp.min(cand, axis=1).astype(jnp.int32)
    nxt = jnp.where(nxt == _E, -1, nxt)

    ys = pl.pallas_call(
        _mlp_body,
        grid_spec=pltpu.PrefetchScalarGridSpec(
            num_scalar_prefetch=5,
            grid=(_NB,),
            in_specs=[
                pl.BlockSpec((_N, _D), lambda i, *_: (0, 0)),
                pl.BlockSpec(memory_space=pl.ANY),
                pl.BlockSpec((1, 1, _F), lambda i, be, *_: (be[i], 0, 0)),
                pl.BlockSpec(memory_space=pl.ANY),
                pl.BlockSpec((1, 1, _D), lambda i, be, *_: (be[i], 0, 0)),
            ],
            out_specs=pl.BlockSpec((_BLKM, _D), lambda i, *_: (i, 0)),
            scratch_shapes=[
                pltpu.VMEM((2, _BLKM, _D), jnp.float32),
                pltpu.VMEM((2, _D, _F), jnp.float32),
                pltpu.VMEM((2, _F, _D), jnp.float32),
                pltpu.SemaphoreType.DMA((2, 2)),
            ],
        ),
        out_shape=jax.ShapeDtypeStruct((_P, _D), jnp.float32),
        compiler_params=pltpu.CompilerParams(dimension_semantics=("arbitrary",)),
    )(block_expert, bufi, is_start, nxt, src_token,
      x, W1, b1.reshape(_E, 1, _F), W2, b2.reshape(_E, 1, _D))

    out = pl.pallas_call(
        _combine_body,
        grid_spec=pltpu.PrefetchScalarGridSpec(
            num_scalar_prefetch=1,
            grid=(_N // _BLK,),
            in_specs=[
                pl.BlockSpec((_BLK, _TOPK), lambda i, pos: (i, 0)),
                pl.BlockSpec((_P, _D), lambda i, pos: (0, 0)),
            ],
            out_specs=pl.BlockSpec((_BLK, _D), lambda i, pos: (i, 0)),
            scratch_shapes=[
                pltpu.VMEM((_BLK, _D), jnp.float32),
                pltpu.VMEM((_BLK, _D), jnp.float32),
            ],
        ),
        out_shape=jax.ShapeDtypeStruct((_N, _D), jnp.float32),
        compiler_params=pltpu.CompilerParams(dimension_semantics=("parallel",)),
    )(slot, w, ys)
    return out
